# baseline (speedup 1.0000x reference)
import jax
import jax.numpy as jnp
from jax import lax
from jax.experimental import pallas as pl
from jax.experimental.pallas import tpu as pltpu
from jax.experimental.pallas import tpu_sc as plsc

BATCH = 1024
SEQ = 512
EMB = 384
OUT = 384
VOCAB = 1000
LANES = 16

NBG = 10
NCG = 3
CW = EMB // NCG
PW = CW // 2
SPW_LO = BATCH // NBG
NBG_HI = BATCH - NBG * SPW_LO
OCH = 48
NCHUNK = OUT // OCH
GPC = OCH // 3
TROWS = 512


def _full16(x):
    return jnp.full((LANES,), x, dtype=jnp.int32)


def _pack_table(table):
    tb = (0.5 * table).astype(jnp.bfloat16)
    bits = jax.lax.bitcast_convert_type(tb, jnp.uint16).astype(jnp.uint32)
    rel = bits.reshape(VOCAB, NCG, PW // LANES, 2, LANES)
    packed = (rel[:, :, :, 1, :] << 16) | rel[:, :, :, 0, :]
    packed = packed.transpose(1, 0, 2, 3).reshape(NCG, VOCAB, PW)
    packed = packed.reshape(NCG, VOCAB // 2, 2 * PW)
    packed = jnp.pad(packed, ((0, 0), (0, TROWS - VOCAB // 2), (0, 0)))
    packed = packed.reshape(NCG, TROWS * 2 * PW)
    return jax.lax.bitcast_convert_type(packed, jnp.int32)


def _sc_body(tokens_hbm, tpack_hbm, out_hbm,
             table_v, tok0, tok1, out0, out1, st0, st1, sw0, sw1):
    info = plsc.get_sparse_core_info()
    wid = lax.axis_index("s") * info.num_cores + lax.axis_index("c")

    @pl.when(wid < NBG * NCG)
    def _active():
        bg = wid // NCG
        cg = lax.rem(wid, NCG)
        b0 = bg * SPW_LO + jnp.minimum(bg, NBG_HI)
        nsamp = SPW_LO + jnp.where(bg < NBG_HI, 1, 0)
        col0 = pl.multiple_of(cg * CW, CW)
        toks = [tok0, tok1]
        sts = [st0, st1]
        outs = [out0, out1]
        sws = [sw0, sw1]
        cols = [lax.iota(jnp.int32, LANES) + (LANES * j)
                for j in range(PW // LANES)]
        mask_hi = jnp.full((LANES,), -65536, dtype=jnp.int32)

        pltpu.sync_copy(tpack_hbm.at[cg], table_v)

        pltpu.sync_copy(tokens_hbm.at[b0], tok0)

        def _sample_body(b, p, tok_v):
            if True:
                @pl.when(b >= 1)
                def _wait_tok():
                    pltpu.make_async_copy(
                        tokens_hbm.at[b0 + b], tok_v, sts[p]).wait()

                @pl.when(b + 1 < nsamp)
                def _prefetch_tok():
                    pltpu.async_copy(
                        tokens_hbm.at[b0 + b + 1], toks[p ^ 1], sts[p ^ 1])

                for c in range(NCHUNK):
                    q = c & 1
                    out_v = outs[q]
                    dst = out_hbm.at[b0 + b, pl.ds(c * OCH, OCH),
                                     pl.ds(col0, CW)]

                    @pl.when(b * NCHUNK + c >= 2)
                    def _drain():
                        pltpu.make_async_copy(out_v, dst, sws[q]).wait()

                    @plsc.parallel_loop(0, GPC, unroll=4)
                    def _group(k):
                        lbase = c * (4 * GPC) + 4 * k
                        t = [plsc.load_gather(tok_v, [_full16(lbase + r)])
                             for r in range(4)]
                        e = []
                        for r in range(4):
                            base = t[r] * PW
                            row = []
                            for j in range(PW // LANES):
                                w = plsc.load_gather(table_v,
                                                     [base + cols[j]])
                                lo = plsc.bitcast(w << 16, jnp.float32)
                                hi = plsc.bitcast(w & mask_hi, jnp.float32)
                                row.append((lo, hi))
                            e.append(row)
                        for r in range(3):
                            orow = 3 * k + r
                            for j in range(PW // LANES):
                                for h in range(2):
                                    out_v[orow,
                                          pl.ds(32 * j + 16 * h, LANES)] = (
                                        e[r][j][h] + e[r + 1][j][h])

                    pltpu.async_copy(out_v, dst, sws[q])

        @pl.loop(0, (nsamp + 1) // 2)
        def _sample_pair(pp):
            for p in range(2):
                b = 2 * pp + p
                tok_v = toks[p]

                @pl.when(b < nsamp)
                def _do_sample():
                    _sample_body(b, p, tok_v)

        for q in range(2):
            pltpu.make_async_copy(
                outs[q], out_hbm.at[0, pl.ds(0, OCH), pl.ds(0, CW)],
                sws[q]).wait()


@jax.jit
def _run(tokens, table):
    tpack = _pack_table(table)
    mesh = plsc.VectorSubcoreMesh(core_axis_name="c", subcore_axis_name="s")
    return pl.kernel(
        _sc_body,
        out_type=jax.ShapeDtypeStruct((BATCH, OUT, EMB), jnp.float32),
        mesh=mesh,
        compiler_params=pltpu.CompilerParams(needs_layout_passes=False),
        scratch_types=[
            pltpu.VMEM((TROWS * 2 * PW,), jnp.int32),
            pltpu.VMEM((SEQ,), jnp.int32),
            pltpu.VMEM((SEQ,), jnp.int32),
            pltpu.VMEM((OCH, CW), jnp.float32),
            pltpu.VMEM((OCH, CW), jnp.float32),
            pltpu.SemaphoreType.DMA,
            pltpu.SemaphoreType.DMA,
            pltpu.SemaphoreType.DMA,
            pltpu.SemaphoreType.DMA,
        ],
    )(tokens, tpack)


def kernel(tokens, table):
    return _run(tokens, table)


# speedup vs baseline: 1.6309x; 1.6309x over previous
import jax
import jax.numpy as jnp
from jax import lax
from jax.experimental import pallas as pl
from jax.experimental.pallas import tpu as pltpu
from jax.experimental.pallas import tpu_sc as plsc

BATCH = 1024
SEQ = 512
EMB = 384
OUT = 384
VOCAB = 1000
LANES = 16

NBG = 10
NCG = 3
CW = EMB // NCG
PW = CW // 2
SPW_LO = BATCH // NBG
NBG_HI = BATCH - NBG * SPW_LO
OCH = 48
NCHUNK = OUT // OCH
GPC = OCH // 3
TROWS = 512


def _full16(x):
    return jnp.full((LANES,), x, dtype=jnp.int32)


def _pack_table(table):
    tb = (0.5 * table).astype(jnp.bfloat16)
    bits = jax.lax.bitcast_convert_type(tb, jnp.uint16).astype(jnp.uint32)
    rel = bits.reshape(VOCAB, NCG, PW // LANES, 2, LANES)
    packed = (rel[:, :, :, 1, :] << 16) | rel[:, :, :, 0, :]
    packed = packed.transpose(1, 0, 2, 3).reshape(NCG, VOCAB, PW)
    packed = packed.reshape(NCG, VOCAB // 2, 2 * PW)
    packed = jnp.pad(packed, ((0, 0), (0, TROWS - VOCAB // 2), (0, 0)))
    packed = packed.reshape(NCG, TROWS * 2 * PW)
    return jax.lax.bitcast_convert_type(packed, jnp.int32)


def _sc_body(tokens_hbm, tpack_hbm, out_hbm,
             table_v, tok0, tok1, out0, out1, st0, st1, sw0, sw1):
    info = plsc.get_sparse_core_info()
    wid = lax.axis_index("s") * info.num_cores + lax.axis_index("c")

    @pl.when(wid < NBG * NCG)
    def _active():
        bg = wid // NCG
        cg = lax.rem(wid, NCG)
        b0 = bg * SPW_LO + jnp.minimum(bg, NBG_HI)
        nsamp = SPW_LO + jnp.where(bg < NBG_HI, 1, 0)
        col0 = pl.multiple_of(cg * CW, CW)
        toks = [tok0, tok1]
        sts = [st0, st1]
        outs = [out0, out1]
        sws = [sw0, sw1]
        cols = [lax.iota(jnp.int32, LANES) + (LANES * j)
                for j in range(PW // LANES)]
        mask_hi = jnp.full((LANES,), -65536, dtype=jnp.int32)

        pltpu.sync_copy(tpack_hbm.at[cg], table_v)

        pltpu.sync_copy(tokens_hbm.at[b0], tok0)

        def _sample_body(b, p, tok_v):
            if True:
                @pl.when(b >= 1)
                def _wait_tok():
                    pltpu.make_async_copy(
                        tokens_hbm.at[b0 + b], tok_v, sts[p]).wait()

                @pl.when(b + 1 < nsamp)
                def _prefetch_tok():
                    pltpu.async_copy(
                        tokens_hbm.at[b0 + b + 1], toks[p ^ 1], sts[p ^ 1])

                for c in range(NCHUNK):
                    q = c & 1
                    out_v = outs[q]
                    dst = out_hbm.at[b0 + b, pl.ds(c * OCH, OCH),
                                     pl.ds(col0, CW)]

                    @pl.when(b * NCHUNK + c >= 2)
                    def _drain():
                        pltpu.make_async_copy(out_v, dst, sws[q]).wait()

                    @plsc.parallel_loop(0, GPC, unroll=2)
                    def _group(k):
                        lbase = c * (4 * GPC) + 4 * k
                        t = [plsc.load_gather(tok_v, [_full16(lbase + r)])
                             for r in range(4)]
                        e = []
                        for r in range(4):
                            base = t[r] * PW
                            row = []
                            for j in range(PW // LANES):
                                w = plsc.load_gather(table_v,
                                                     [base + cols[j]])
                                lo = plsc.bitcast(w << 16, jnp.float32)
                                hi = plsc.bitcast(w & mask_hi, jnp.float32)
                                row.append((lo, hi))
                            e.append(row)
                        for r in range(3):
                            orow = 3 * k + r
                            for j in range(PW // LANES):
                                for h in range(2):
                                    out_v[orow,
                                          pl.ds(32 * j + 16 * h, LANES)] = (
                                        e[r][j][h] + e[r + 1][j][h])

                    pltpu.async_copy(out_v, dst, sws[q])

        @pl.loop(0, (nsamp + 1) // 2)
        def _sample_pair(pp):
            for p in range(2):
                b = 2 * pp + p
                tok_v = toks[p]

                @pl.when(b < nsamp)
                def _do_sample():
                    _sample_body(b, p, tok_v)

        for q in range(2):
            pltpu.make_async_copy(
                outs[q], out_hbm.at[0, pl.ds(0, OCH), pl.ds(0, CW)],
                sws[q]).wait()


@jax.jit
def _run(tokens, table):
    tpack = _pack_table(table)
    mesh = plsc.VectorSubcoreMesh(core_axis_name="c", subcore_axis_name="s")
    return pl.kernel(
        _sc_body,
        out_type=jax.ShapeDtypeStruct((BATCH, OUT, EMB), jnp.float32),
        mesh=mesh,
        compiler_params=pltpu.CompilerParams(needs_layout_passes=False),
        scratch_types=[
            pltpu.VMEM((TROWS * 2 * PW,), jnp.int32),
            pltpu.VMEM((SEQ,), jnp.int32),
            pltpu.VMEM((SEQ,), jnp.int32),
            pltpu.VMEM((OCH, CW), jnp.float32),
            pltpu.VMEM((OCH, CW), jnp.float32),
            pltpu.SemaphoreType.DMA,
            pltpu.SemaphoreType.DMA,
            pltpu.SemaphoreType.DMA,
            pltpu.SemaphoreType.DMA,
        ],
    )(tokens, tpack)


def kernel(tokens, table):
    return _run(tokens, table)
